# baseline (speedup 1.0000x reference)
import jax
import jax.numpy as jnp
from jax import lax
from jax.experimental import pallas as pl
from jax.experimental.pallas import tpu as pltpu
from jax.experimental.pallas import tpu_sc as plsc

VOCAB = 1000
EMB_DIM = 64
B = 4096
T = 200
NC, NS = 2, 16
NW = NC * NS
BL = 128
DT = EMB_DIM // 8
TT = T // 8
TS = BL + 4


def _emb_lookup(idx5, table):
    mesh = plsc.VectorSubcoreMesh(core_axis_name="c", subcore_axis_name="s")

    @pl.kernel(
        mesh=mesh,
        out_type=jax.ShapeDtypeStruct((T, DT, NW, 8, BL), jnp.float32),
        scratch_types=[
            pltpu.VMEM((T, BL), jnp.int32),
            pltpu.VMEM((BL, EMB_DIM), jnp.float32),
            pltpu.VMEM((BL, EMB_DIM), jnp.float32),
            pltpu.VMEM((2, EMB_DIM, TS), jnp.float32),
            pltpu.SemaphoreType.DMA,
            pltpu.SemaphoreType.DMA,
            pltpu.SemaphoreType.DMA,
            pltpu.SemaphoreType.DMA,
        ],
        compiler_params=pltpu.CompilerParams(
            use_tc_tiling_on_sc=False, needs_layout_passes=False
        ),
    )
    def k(idx_hbm, tab_hbm, out_hbm, idx_v, g0, g1, tbuf, gs0, gs1, s0, s1):
        w = lax.axis_index("s") * NC + lax.axis_index("c")
        gbufs = (g0, g1)
        gsem = (gs0, gs1)
        ssem = (s0, s1)
        for tt in range(TT):
            pltpu.sync_copy(idx_hbm.at[tt, w], idx_v.at[pl.ds(tt * 8, 8)])

        iota = lax.iota(jnp.int32, 16)
        rot = iota * 4
        rvecs = [iota + r0 for r0 in range(0, BL, 16)]

        def fire_gather(t, gi):
            pltpu.async_copy(tab_hbm.at[idx_v.at[t]], gbufs[gi], gsem[gi])

        def wait_gather(gi):
            pltpu.make_async_copy(
                tab_hbm.at[pl.ds(0, BL)], gbufs[gi], gsem[gi]
            ).wait()

        def transpose(gi):
            gb = gbufs[gi]
            tb = tbuf.at[gi]

            def jbody(jq, carry):
                for jj in range(4):
                    j = jq * 4 + jj
                    dvec = lax.bitwise_and(rot + j, 63)
                    for g16 in range(8):
                        val = plsc.load_gather(gb, [rvecs[g16], dvec])
                        plsc.store_scatter(tb, [dvec, rvecs[g16]], val)
                return carry

            lax.fori_loop(0, EMB_DIM // 4, jbody, 0)

        def fire_scatter(t, bi):
            for dt in range(DT):
                pltpu.async_copy(
                    tbuf.at[bi].at[pl.ds(dt * 8, 8), pl.ds(0, BL)],
                    out_hbm.at[t, dt, w],
                    ssem[bi],
                )

        def wait_scatter(bi):
            for dt in range(DT):
                pltpu.make_async_copy(
                    tbuf.at[bi].at[pl.ds(dt * 8, 8), pl.ds(0, BL)],
                    out_hbm.at[0, 0, 0],
                    ssem[bi],
                ).wait()

        fire_gather(0, 0)
        fire_gather(1, 1)

        def body(i, carry):
            for gi in range(2):
                t = i * 2 + gi
                wait_gather(gi)

                @pl.when(t >= 2)
                def _():
                    wait_scatter(gi)

                transpose(gi)

                @pl.when(t + 2 < T)
                def _():
                    fire_gather(t + 2, gi)

                fire_scatter(t, gi)
            return carry

        lax.fori_loop(0, T // 2, body, 0)
        wait_scatter(0)
        wait_scatter(1)

    return k(idx5, table)


def kernel(indices, embedding_weight):
    idx5 = (
        jnp.asarray(indices, jnp.int32)
        .reshape(NW, BL, TT, 8)
        .transpose(2, 0, 3, 1)
    )
    out5 = _emb_lookup(idx5, embedding_weight)
    return out5.transpose(2, 4, 0, 1, 3).reshape(B, T, EMB_DIM)


# speedup vs baseline: 1.0654x; 1.0654x over previous
import jax
import jax.numpy as jnp
from jax import lax
from jax.experimental import pallas as pl
from jax.experimental.pallas import tpu as pltpu
from jax.experimental.pallas import tpu_sc as plsc

VOCAB = 1000
EMB_DIM = 64
B = 4096
T = 200
NC, NS = 2, 16
NW = NC * NS
BL = 128
DT = EMB_DIM // 8
TT = T // 8
TS = BL + 4


def _emb_lookup(idx5, table):
    mesh = plsc.VectorSubcoreMesh(core_axis_name="c", subcore_axis_name="s")

    @pl.kernel(
        mesh=mesh,
        out_type=jax.ShapeDtypeStruct((T, DT, NW, 8, BL), jnp.float32),
        scratch_types=[
            pltpu.VMEM((T, BL), jnp.int32),
            pltpu.VMEM((BL, EMB_DIM), jnp.float32),
            pltpu.VMEM((BL, EMB_DIM), jnp.float32),
            pltpu.VMEM((2, EMB_DIM, TS), jnp.float32),
            pltpu.SemaphoreType.DMA,
            pltpu.SemaphoreType.DMA,
            pltpu.SemaphoreType.DMA,
            pltpu.SemaphoreType.DMA,
        ],
        compiler_params=pltpu.CompilerParams(
            use_tc_tiling_on_sc=False, needs_layout_passes=False
        ),
    )
    def k(idx_hbm, tab_hbm, out_hbm, idx_v, g0, g1, tbuf, gs0, gs1, s0, s1):
        w = lax.axis_index("s") * NC + lax.axis_index("c")
        gbufs = (g0, g1)
        gsem = (gs0, gs1)
        ssem = (s0, s1)
        for tt in range(TT):
            pltpu.sync_copy(idx_hbm.at[tt, w], idx_v.at[pl.ds(tt * 8, 8)])

        iota = lax.iota(jnp.int32, 16)
        rot = iota * 4
        rvecs = [iota + r0 for r0 in range(0, BL, 16)]

        def fire_gather(t, gi):
            pltpu.async_copy(tab_hbm.at[idx_v.at[t]], gbufs[gi], gsem[gi])

        def wait_gather(gi):
            pltpu.make_async_copy(
                tab_hbm.at[pl.ds(0, BL)], gbufs[gi], gsem[gi]
            ).wait()

        def transpose(gi):
            gb = gbufs[gi]
            tb = tbuf.at[gi]

            def loads(j):
                dvec = lax.bitwise_and(rot + j, 63)
                return dvec, [
                    plsc.load_gather(gb, [rvecs[g16], dvec])
                    for g16 in range(8)
                ]

            def stores(dvec, vals):
                for g16 in range(8):
                    plsc.store_scatter(tb, [dvec, rvecs[g16]], vals[g16])

            def jbody(jq, carry):
                prev = loads(jq * 8)
                for jj in range(1, 8):
                    cur = loads(jq * 8 + jj)
                    stores(*prev)
                    prev = cur
                stores(*prev)
                return carry

            lax.fori_loop(0, EMB_DIM // 8, jbody, 0)

        def fire_scatter(t, bi):
            for dt in range(DT):
                pltpu.async_copy(
                    tbuf.at[bi].at[pl.ds(dt * 8, 8), pl.ds(0, BL)],
                    out_hbm.at[t, dt, w],
                    ssem[bi],
                )

        def wait_scatter(bi):
            for dt in range(DT):
                pltpu.make_async_copy(
                    tbuf.at[bi].at[pl.ds(dt * 8, 8), pl.ds(0, BL)],
                    out_hbm.at[0, 0, 0],
                    ssem[bi],
                ).wait()

        fire_gather(0, 0)
        fire_gather(1, 1)

        def body(i, carry):
            for gi in range(2):
                t = i * 2 + gi
                wait_gather(gi)

                @pl.when(t >= 2)
                def _():
                    wait_scatter(gi)

                transpose(gi)

                @pl.when(t + 2 < T)
                def _():
                    fire_gather(t + 2, gi)

                fire_scatter(t, gi)
            return carry

        lax.fori_loop(0, T // 2, body, 0)
        wait_scatter(0)
        wait_scatter(1)

    return k(idx5, table)


def kernel(indices, embedding_weight):
    idx5 = (
        jnp.asarray(indices, jnp.int32)
        .reshape(NW, BL, TT, 8)
        .transpose(2, 0, 3, 1)
    )
    out5 = _emb_lookup(idx5, embedding_weight)
    return out5.transpose(2, 4, 0, 1, 3).reshape(B, T, EMB_DIM)


# speedup vs baseline: 1.2399x; 1.1638x over previous
import jax
import jax.numpy as jnp
from jax import lax
from jax.experimental import pallas as pl
from jax.experimental.pallas import tpu as pltpu
from jax.experimental.pallas import tpu_sc as plsc

VOCAB = 1000
EMB_DIM = 64
B = 4096
T = 200
NC, NS = 2, 16
NW = NC * NS
BL = 128
DT = EMB_DIM // 8
TT = T // 8
TS = BL + 4


def _emb_lookup(idx5, table):
    mesh = plsc.VectorSubcoreMesh(core_axis_name="c", subcore_axis_name="s")

    @pl.kernel(
        mesh=mesh,
        out_type=jax.ShapeDtypeStruct((T, DT, NW, 8, BL), jnp.float32),
        scratch_types=[
            pltpu.VMEM((T, BL), jnp.int32),
            pltpu.VMEM((BL, EMB_DIM), jnp.float32),
            pltpu.VMEM((BL, EMB_DIM), jnp.float32),
            pltpu.VMEM((2, DT, 8, TS), jnp.float32),
            pltpu.SemaphoreType.DMA,
            pltpu.SemaphoreType.DMA,
            pltpu.SemaphoreType.DMA,
            pltpu.SemaphoreType.DMA,
        ],
        compiler_params=pltpu.CompilerParams(
            use_tc_tiling_on_sc=False, needs_layout_passes=False
        ),
    )
    def k(idx_hbm, tab_hbm, out_hbm, idx_v, g0, g1, tbuf, gs0, gs1, s0, s1):
        w = lax.axis_index("s") * NC + lax.axis_index("c")
        gbufs = (g0, g1)
        gsem = (gs0, gs1)
        ssem = (s0, s1)
        for tt in range(TT):
            pltpu.sync_copy(idx_hbm.at[tt, w], idx_v.at[pl.ds(tt * 8, 8)])

        iota = lax.iota(jnp.int32, 16)
        rot = iota * 4
        rvecs = [iota + r0 for r0 in range(0, BL, 16)]

        def fire_gather(t, gi):
            pltpu.async_copy(tab_hbm.at[idx_v.at[t]], gbufs[gi], gsem[gi])

        def wait_gather(gi):
            pltpu.make_async_copy(
                tab_hbm.at[pl.ds(0, BL)], gbufs[gi], gsem[gi]
            ).wait()

        def transpose(gi):
            gb = gbufs[gi]
            tb = tbuf.at[gi]

            def loads(j):
                dvec = lax.bitwise_and(rot + j, 63)
                return dvec, [
                    plsc.load_gather(gb, [rvecs[g16], dvec])
                    for g16 in range(8)
                ]

            def stores(dvec, vals):
                dtv = lax.shift_right_logical(dvec, 3)
                dsv = lax.bitwise_and(dvec, 7)
                for g16 in range(8):
                    plsc.store_scatter(tb, [dtv, dsv, rvecs[g16]], vals[g16])

            def jbody(jq, carry):
                prev = loads(jq * 8)
                for jj in range(1, 8):
                    cur = loads(jq * 8 + jj)
                    stores(*prev)
                    prev = cur
                stores(*prev)
                return carry

            lax.fori_loop(0, EMB_DIM // 8, jbody, 0)

        def fire_scatter(t, bi):
            pltpu.async_copy(
                tbuf.at[bi].at[:, :, pl.ds(0, BL)],
                out_hbm.at[t, :, w],
                ssem[bi],
            )

        def wait_scatter(bi):
            pltpu.make_async_copy(
                tbuf.at[bi].at[:, :, pl.ds(0, BL)],
                out_hbm.at[0, :, 0],
                ssem[bi],
            ).wait()

        fire_gather(0, 0)
        fire_gather(1, 1)

        def body(i, carry):
            for gi in range(2):
                t = i * 2 + gi
                wait_gather(gi)

                @pl.when(t >= 2)
                def _():
                    wait_scatter(gi)

                transpose(gi)

                @pl.when(t + 2 < T)
                def _():
                    fire_gather(t + 2, gi)

                fire_scatter(t, gi)
            return carry

        lax.fori_loop(0, T // 2, body, 0)
        wait_scatter(0)
        wait_scatter(1)

    return k(idx5, table)


def kernel(indices, embedding_weight):
    idx5 = (
        jnp.asarray(indices, jnp.int32)
        .reshape(NW, BL, TT, 8)
        .transpose(2, 0, 3, 1)
    )
    out5 = _emb_lookup(idx5, embedding_weight)
    return out5.transpose(2, 4, 0, 1, 3).reshape(B, T, EMB_DIM)


# speedup vs baseline: 1.4626x; 1.1796x over previous
import jax
import jax.numpy as jnp
from jax import lax
from jax.experimental import pallas as pl
from jax.experimental.pallas import tpu as pltpu
from jax.experimental.pallas import tpu_sc as plsc

VOCAB = 1000
EMB_DIM = 64
B = 4096
T = 200
NC, NS = 2, 16
NW = NC * NS
BL = 128
DT = EMB_DIM // 8
TT = T // 8
TS = BL + 4
CR = 32
RS = CR + 1


def _emb_lookup(idx5, tab_res, tab_str):
    mesh = plsc.VectorSubcoreMesh(core_axis_name="c", subcore_axis_name="s")

    @pl.kernel(
        mesh=mesh,
        out_type=jax.ShapeDtypeStruct((T, DT, NW, 8, BL), jnp.float32),
        scratch_types=[
            pltpu.VMEM((T, BL), jnp.int32),
            pltpu.VMEM((VOCAB * RS,), jnp.float32),
            pltpu.VMEM((BL, CR), jnp.float32),
            pltpu.VMEM((BL, CR), jnp.float32),
            pltpu.VMEM((2, DT, 8, TS), jnp.float32),
            pltpu.SemaphoreType.DMA,
            pltpu.SemaphoreType.DMA,
            pltpu.SemaphoreType.DMA,
            pltpu.SemaphoreType.DMA,
        ],
        compiler_params=pltpu.CompilerParams(
            use_tc_tiling_on_sc=False, needs_layout_passes=False
        ),
    )
    def k(idx_hbm, res_hbm, str_hbm, out_hbm, idx_v, res_v, g0, g1, tbuf,
          gs0, gs1, s0, s1):
        w = lax.axis_index("s") * NC + lax.axis_index("c")
        gbufs = (g0, g1)
        gsem = (gs0, gs1)
        ssem = (s0, s1)
        pltpu.sync_copy(res_hbm, res_v)
        for tt in range(TT):
            pltpu.sync_copy(idx_hbm.at[tt, w], idx_v.at[pl.ds(tt * 8, 8)])

        iota = lax.iota(jnp.int32, 16)
        rot = iota * 4
        rvecs = [iota + r0 for r0 in range(0, BL, 16)]

        def fire_gather(t, gi):
            pltpu.async_copy(str_hbm.at[idx_v.at[t]], gbufs[gi], gsem[gi])

        def wait_gather(gi):
            pltpu.make_async_copy(
                str_hbm.at[pl.ds(0, BL)], gbufs[gi], gsem[gi]
            ).wait()

        def compute_t(t, gi):
            gb = gbufs[gi]
            tb = tbuf.at[gi]
            bases = []
            for g16 in range(8):
                iv = idx_v[t, pl.ds(g16 * 16, 16)]
                bases.append(iv * RS)

            def loads(j):
                dvec = lax.bitwise_and(rot + j, CR - 1)
                res = [
                    plsc.load_gather(res_v, [bases[g16] + dvec])
                    for g16 in range(8)
                ]
                stream = [
                    plsc.load_gather(gb, [rvecs[g16], dvec])
                    for g16 in range(8)
                ]
                return dvec, res, stream

            def stores(dvec, res, stream):
                dtv = lax.shift_right_logical(dvec, 3)
                dsv = lax.bitwise_and(dvec, 7)
                for g16 in range(8):
                    plsc.store_scatter(tb, [dtv, dsv, rvecs[g16]], res[g16])
                dtv2 = dtv + CR // 8
                for g16 in range(8):
                    plsc.store_scatter(
                        tb, [dtv2, dsv, rvecs[g16]], stream[g16]
                    )

            def jbody(jq, carry):
                prev = loads(jq * 8)
                for jj in range(1, 8):
                    cur = loads(jq * 8 + jj)
                    stores(*prev)
                    prev = cur
                stores(*prev)
                return carry

            lax.fori_loop(0, CR // 8, jbody, 0)

        def fire_scatter(t, bi):
            pltpu.async_copy(
                tbuf.at[bi].at[:, :, pl.ds(0, BL)],
                out_hbm.at[t, :, w],
                ssem[bi],
            )

        def wait_scatter(bi):
            pltpu.make_async_copy(
                tbuf.at[bi].at[:, :, pl.ds(0, BL)],
                out_hbm.at[0, :, 0],
                ssem[bi],
            ).wait()

        fire_gather(0, 0)
        fire_gather(1, 1)

        def body(i, carry):
            for gi in range(2):
                t = i * 2 + gi
                wait_gather(gi)

                @pl.when(t >= 2)
                def _():
                    wait_scatter(gi)

                compute_t(t, gi)

                @pl.when(t + 2 < T)
                def _():
                    fire_gather(t + 2, gi)

                fire_scatter(t, gi)
            return carry

        lax.fori_loop(0, T // 2, body, 0)
        wait_scatter(0)
        wait_scatter(1)

    return k(idx5, tab_res, tab_str)


def kernel(indices, embedding_weight):
    idx5 = (
        jnp.asarray(indices, jnp.int32)
        .reshape(NW, BL, TT, 8)
        .transpose(2, 0, 3, 1)
    )
    tab_res = jnp.pad(embedding_weight[:, :CR], ((0, 0), (0, RS - CR)))
    out5 = _emb_lookup(idx5, tab_res.reshape(-1), embedding_weight[:, CR:])
    return out5.transpose(2, 4, 0, 1, 3).reshape(B, T, EMB_DIM)
